# Initial kernel scaffold; baseline (speedup 1.0000x reference)
import jax
import jax.numpy as jnp
from jax.experimental import pallas as pl


def kernel(x_0, x_1, incidence_rows, incidence_cols, W0_in, b0_in, W1_in, b1_in, W0_out, b0_out, W1_out, b1_out):
    raise NotImplementedError("write your pallas kernel here")


# speedup vs baseline: 4.1280x; 4.1280x over previous
import functools

import jax
import jax.numpy as jnp
from jax import lax
from jax.experimental import pallas as pl
from jax.experimental.pallas import tpu as pltpu
from jax.experimental.pallas import tpu_sc as plsc

_N0, _N1, _NNZ, _D0, _H = 50000, 25000, 800000, 8, 64
_N0P = 50176
_N1P = 25088
_PK1 = _N1P // 2
_PK0 = _N0P // 2
_BSP = 12544
_CW = 128
_NNZP = 819200
_IDXR = _NNZP // _CW
_SLAB = 8
_ZR = 16


def _sc_body(rows_hbm, cols_hbm, tbl_hbm, parts, bufv, bufe, feat, zb, acc,
             *, phase):
    c = lax.axis_index("c")
    s = lax.axis_index("s")

    def _zrow(r, carry):
        for k in range(8):
            zb[r, pl.ds(16 * k, 16)] = jnp.zeros((16,), jnp.float32)
        return carry
    lax.fori_loop(0, _ZR, _zrow, 0)
    nacc = acc.shape[0] // 16
    zbase = s * nacc
    for t in range(nacc // _ZR):
        pltpu.sync_copy(zb, acc.at[pl.ds(zbase + t * _ZR, _ZR)])
    plsc.subcore_barrier()

    nslab = 25 if phase == 0 else 50
    wbase = (c * 16 + s) * 200 if phase == 0 else s * 400
    iota = lax.broadcasted_iota(jnp.int32, (16,), 0)

    def _slab(t, carry):
        start = wbase + t * _SLAB
        pltpu.sync_copy(rows_hbm.at[pl.ds(start, _SLAB)], bufv)
        pltpu.sync_copy(cols_hbm.at[pl.ds(start, _SLAB)], bufe)

        def _tr(r, carry2):
            for k in range(8):
                sl = pl.ds(16 * k, 16)
                v = bufv[r, sl]
                e = bufe[r, sl]
                pad = v >= _N0
                spread = r * 128 + k * 16 + iota
                if phase == 0:
                    hi = e >= _PK1
                    bufv[r, sl] = jnp.where(
                        pad, spread, v + jnp.where(hi, _N0P, 0))
                    bufe[r, sl] = jnp.where(
                        pad, -1, e - jnp.where(hi, _PK1, 0))
                else:
                    reali = -((v - _N0) >> 31)
                    hivi = 1 + ((v - _PK0) >> 31)
                    bufv[r, sl] = (reali * (e + hivi * _N1P)
                                   + (1 - reali) * spread)
                    loc = v - hivi * _PK0 - c * _BSP
                    ge0i = 1 + (loc >> 31)
                    lti = -((loc - _BSP) >> 31)
                    owni = reali * ge0i * lti
                    bufe[r, sl] = owni * (loc + 1) - 1
            return carry2
        lax.fori_loop(0, _SLAB, _tr, 0)

        def _chunk(j, carry2):
            pltpu.sync_copy(tbl_hbm.at[bufv.at[j]], feat)
            pltpu.sync_copy(
                feat, acc.at[plsc.Indices(bufe.at[j], ignored_value=-1)],
                add=True)
            return carry2
        lax.fori_loop(0, _SLAB, _chunk, 0)
        return carry
    lax.fori_loop(0, nslab, _slab, 0)
    plsc.subcore_barrier()

    pltpu.sync_copy(acc.at[pl.ds(zbase, nacc)], parts.at[c, pl.ds(zbase, nacc)])


def _make_sc(phase, acc_rows):
    mesh = plsc.VectorSubcoreMesh(core_axis_name="c", subcore_axis_name="s",
                                  num_cores=2, num_subcores=16)
    out_type = [jax.ShapeDtypeStruct((2, acc_rows, 128), jnp.float32)]
    scratch = [
        pltpu.VMEM((_SLAB, _CW), jnp.int32),
        pltpu.VMEM((_SLAB, _CW), jnp.int32),
        pltpu.VMEM((_CW, 128), jnp.float32),
        pltpu.VMEM((_ZR, 128), jnp.float32),
        pltpu.VMEM_SHARED((acc_rows, 128), jnp.float32),
    ]
    body = functools.partial(_sc_body, phase=phase)
    return pl.kernel(body, out_type=out_type, mesh=mesh,
                     scratch_types=scratch)


def _deg_body(rows_hbm, deg0, deg1, buf, ones, zd, dacc):
    c = lax.axis_index("c")
    s = lax.axis_index("s")

    def _zo(r, carry):
        ones[pl.ds(16 * r, 16)] = jnp.ones((16,), jnp.float32)
        return carry
    lax.fori_loop(0, _CW // 16, _zo, 0)

    def _zd(r, carry):
        zd[pl.ds(16 * r, 16)] = jnp.zeros((16,), jnp.float32)
        return carry
    lax.fori_loop(0, 98, _zd, 0)
    nd = _N0P // 16
    pltpu.sync_copy(zd, dacc.at[pl.ds(s * nd, 1568)])
    pltpu.sync_copy(zd, dacc.at[pl.ds(s * nd + 1568, 1568)])
    plsc.subcore_barrier()

    wbase = (c * 16 + s) * 200

    def _slab(t, carry):
        pltpu.sync_copy(rows_hbm.at[pl.ds(wbase + t * _SLAB, _SLAB)], buf)

        def _chunk(j, carry2):
            pltpu.sync_copy(ones, dacc.at[buf.at[j]], add=True)
            return carry2
        lax.fori_loop(0, _SLAB, _chunk, 0)
        return carry
    lax.fori_loop(0, 25, _slab, 0)
    plsc.subcore_barrier()

    @pl.when(c == 0)
    def _():
        pltpu.sync_copy(dacc.at[pl.ds(s * nd, 1568)], zd)
        pltpu.sync_copy(zd, deg0.at[pl.ds(s * nd, 1568)])
        pltpu.sync_copy(dacc.at[pl.ds(s * nd + 1568, 1568)], zd)
        pltpu.sync_copy(zd, deg0.at[pl.ds(s * nd + 1568, 1568)])

    @pl.when(c == 1)
    def _():
        pltpu.sync_copy(dacc.at[pl.ds(s * nd, 1568)], zd)
        pltpu.sync_copy(zd, deg1.at[pl.ds(s * nd, 1568)])
        pltpu.sync_copy(dacc.at[pl.ds(s * nd + 1568, 1568)], zd)
        pltpu.sync_copy(zd, deg1.at[pl.ds(s * nd + 1568, 1568)])


def _make_deg():
    mesh = plsc.VectorSubcoreMesh(core_axis_name="c", subcore_axis_name="s",
                                  num_cores=2, num_subcores=16)
    out_type = [jax.ShapeDtypeStruct((_N0P,), jnp.float32),
                jax.ShapeDtypeStruct((_N0P,), jnp.float32)]
    scratch = [
        pltpu.VMEM((_SLAB, _CW), jnp.int32),
        pltpu.VMEM((_CW,), jnp.float32),
        pltpu.VMEM((1568,), jnp.float32),
        pltpu.VMEM_SHARED((_N0P,), jnp.float32),
    ]
    return pl.kernel(_deg_body, out_type=out_type, mesh=mesh,
                     scratch_types=scratch)


def _proj_body(x_ref, w_ref, b_ref, o_ref):
    th = pl.program_id(0)
    h = jnp.dot(x_ref[...], w_ref[...],
                preferred_element_type=jnp.float32) + b_ref[...]
    z = jnp.zeros_like(h)
    o_ref[...] = jnp.where(th == 0,
                           jnp.concatenate([h, z], axis=1),
                           jnp.concatenate([z, h], axis=1))[None]


def _proj_build(x0p, W, b):
    return pl.pallas_call(
        _proj_body,
        grid=(2, _N0P // 512),
        in_specs=[pl.BlockSpec((512, _D0), lambda t, g: (g, 0)),
                  pl.BlockSpec((_D0, _H), lambda t, g: (0, 0)),
                  pl.BlockSpec((1, _H), lambda t, g: (0, 0))],
        out_specs=pl.BlockSpec((1, 512, 128), lambda t, g: (t, g, 0)),
        out_shape=jax.ShapeDtypeStruct((2, _N0P, 128), jnp.float32),
    )(x0p, W, b)


def _comb_body(p_ref, o_ref):
    th = pl.program_id(0)
    ch = pl.program_id(1)
    S = p_ref[0] + p_ref[1]
    h = jnp.where(ch == 0, S[:, :_H], S[:, _H:])
    z = jnp.zeros_like(h)
    o_ref[...] = jnp.where(th == 0,
                           jnp.concatenate([h, z], axis=1),
                           jnp.concatenate([z, h], axis=1))[None]


def _combine_build(parts):
    blk = 448
    nb = _PK1 // blk
    return pl.pallas_call(
        _comb_body,
        grid=(2, 2, nb),
        in_specs=[pl.BlockSpec((2, blk, 128), lambda t, ch, g: (0, g, 0))],
        out_specs=pl.BlockSpec((1, blk, 128),
                               lambda t, ch, g: (t, ch * nb + g, 0)),
        out_shape=jax.ShapeDtypeStruct((2, _N1P, 128), jnp.float32),
    )(parts)


def _upd_body(pb_ref, t0_ref, d0_ref, d1_ref, o_ref):
    th = pl.program_id(0)
    vh = pl.program_id(1) // 56
    S = pb_ref[0]
    m = jnp.where(vh == 0, S[:, :_H], S[:, _H:])
    d = jnp.maximum(d0_ref[...] + d1_ref[...], 1.0)
    h = t0_ref[0, :, :_H] + m / d
    z = jnp.zeros_like(h)
    o_ref[...] = jnp.where(th == 0,
                           jnp.concatenate([h, z], axis=1),
                           jnp.concatenate([z, h], axis=1))[None]


def _update_build(partsb, T0, deg0, deg1):
    nbs = _BSP // 448

    def _pb_idx(t, g):
        gq = g % 56
        part = gq // nbs
        return (part, gq - part * nbs, 0)

    dspec = pl.BlockSpec((448, 1), lambda t, g: (g, 0))
    return pl.pallas_call(
        _upd_body,
        grid=(2, 112),
        in_specs=[pl.BlockSpec((1, 448, 128), _pb_idx),
                  pl.BlockSpec((1, 448, 128), lambda t, g: (0, g, 0)),
                  dspec, dspec],
        out_specs=pl.BlockSpec((1, 448, 128), lambda t, g: (t, g, 0)),
        out_shape=jax.ShapeDtypeStruct((2, _N0P, 128), jnp.float32),
    )(partsb, T0, deg0.reshape(_N0P, 1), deg1.reshape(_N0P, 1))


def _fin_body(t0_ref, t1_ref, w0_ref, b0_ref, w1_ref, b1_ref, o_ref,
              acc0, acc1):
    g = pl.program_id(0)

    @pl.when(g == 0)
    def _():
        acc0[...] = jnp.zeros_like(acc0)
        acc1[...] = jnp.zeros_like(acc1)

    b0r = _N0P // 32
    r0 = lax.broadcasted_iota(jnp.int32, (b0r, 128), 0) + g * b0r
    acc0[...] += jnp.sum(jnp.where(r0 < _N0, t0_ref[0], 0.0),
                         axis=0, keepdims=True)
    acc1[...] += jnp.sum(t1_ref[0], axis=0, keepdims=True)

    @pl.when(g == 31)
    def _():
        s0 = acc0[:, :_H] * (1.0 / _N0)
        s1 = acc1[:, :_H] * (1.0 / _N1)
        o_ref[...] = (jnp.dot(s0, w0_ref[...],
                              preferred_element_type=jnp.float32) + b0_ref[...]
                      + jnp.dot(s1, w1_ref[...],
                                preferred_element_type=jnp.float32)
                      + b1_ref[...])


def _final(T0, T1, W0o, b0o, W1o, b1o):
    return pl.pallas_call(
        _fin_body,
        grid=(32,),
        in_specs=[pl.BlockSpec((1, _N0P // 32, 128), lambda g: (0, g, 0)),
                  pl.BlockSpec((1, _N1P // 32, 128), lambda g: (0, g, 0)),
                  pl.BlockSpec((_H, 1), lambda g: (0, 0)),
                  pl.BlockSpec((1, 1), lambda g: (0, 0)),
                  pl.BlockSpec((_H, 1), lambda g: (0, 0)),
                  pl.BlockSpec((1, 1), lambda g: (0, 0))],
        out_specs=pl.BlockSpec((1, 1), lambda g: (0, 0)),
        out_shape=jax.ShapeDtypeStruct((1, 1), jnp.float32),
        scratch_shapes=[pltpu.VMEM((1, 128), jnp.float32),
                        pltpu.VMEM((1, 128), jnp.float32)],
    )(T0, T1, W0o, b0o, W1o, b1o)


def kernel(x_0, x_1, incidence_rows, incidence_cols,
           W0_in, b0_in, W1_in, b1_in, W0_out, b0_out, W1_out, b1_out):
    del x_1, W1_in, b1_in
    npad = _NNZP - _NNZ
    rows_p = jnp.concatenate(
        [incidence_rows, jnp.full((npad,), _N0, jnp.int32)]).reshape(
        _IDXR, _CW)
    cols_p = jnp.concatenate(
        [incidence_cols, jnp.full((npad,), _N1, jnp.int32)]).reshape(
        _IDXR, _CW)
    x0p = jnp.concatenate([x_0, jnp.zeros((_N0P - _N0, _D0), jnp.float32)])

    T0 = _proj_build(x0p, W0_in, b0_in.reshape(1, _H))
    deg0, deg1 = _make_deg()(rows_p)

    phase_a = _make_sc(phase=0, acc_rows=_PK1)
    phase_b = _make_sc(phase=1, acc_rows=_BSP)

    parts1, = phase_a(rows_p, cols_p, T0.reshape(2 * _N0P, 128))
    T1 = _combine_build(parts1)
    partsb, = phase_b(rows_p, cols_p, T1.reshape(2 * _N1P, 128))
    T0 = _update_build(partsb, T0, deg0, deg1)

    parts2, = phase_a(rows_p, cols_p, T0.reshape(2 * _N0P, 128))
    T1 = _combine_build(parts2)
    partsb, = phase_b(rows_p, cols_p, T1.reshape(2 * _N1P, 128))
    T0 = _update_build(partsb, T0, deg0, deg1)

    out = _final(T0, T1, W0_out, b0_out.reshape(1, 1),
                 W1_out, b1_out.reshape(1, 1))
    return out.reshape(1)


# speedup vs baseline: 5.5468x; 1.3437x over previous
import functools

import jax
import jax.numpy as jnp
from jax import lax
from jax.experimental import pallas as pl
from jax.experimental.pallas import tpu as pltpu
from jax.experimental.pallas import tpu_sc as plsc

_N0, _N1, _NNZ, _D0, _H = 50000, 25000, 800000, 8, 64
_N0P = 50176
_N1P = 25088
_PK1 = _N1P // 2
_PK0 = _N0P // 2
_BSP = 12544
_CW = 128
_NNZP = 819200
_IDXR = _NNZP // _CW
_SLAB = 8
_ZR = 16


def _sc_body(rows_hbm, cols_hbm, tbl_hbm, parts, bufv, bufe, feat, zb, acc,
             *, phase):
    c = lax.axis_index("c")
    s = lax.axis_index("s")

    def _zrow(r, carry):
        for k in range(8):
            zb[r, pl.ds(16 * k, 16)] = jnp.zeros((16,), jnp.float32)
        return carry
    lax.fori_loop(0, _ZR, _zrow, 0)
    nacc = acc.shape[0] // 16
    zbase = s * nacc
    for t in range(nacc // _ZR):
        pltpu.sync_copy(zb, acc.at[pl.ds(zbase + t * _ZR, _ZR)])
    plsc.subcore_barrier()

    nslab = 25 if phase == 0 else 50
    wbase = (c * 16 + s) * 200 if phase == 0 else s * 400
    iota = lax.broadcasted_iota(jnp.int32, (16,), 0)

    def _slab(t, carry):
        start = wbase + t * _SLAB
        pltpu.sync_copy(rows_hbm.at[pl.ds(start, _SLAB)], bufv)
        pltpu.sync_copy(cols_hbm.at[pl.ds(start, _SLAB)], bufe)

        def _tr(r, carry2):
            for k in range(8):
                sl = pl.ds(16 * k, 16)
                v = bufv[r, sl]
                e = bufe[r, sl]
                pad = v >= _N0
                spread = r * 128 + k * 16 + iota
                if phase == 0:
                    hi = e >= _PK1
                    bufv[r, sl] = jnp.where(
                        pad, spread, v + jnp.where(hi, _N0P, 0))
                    bufe[r, sl] = jnp.where(
                        pad, -1, e - jnp.where(hi, _PK1, 0))
                else:
                    reali = -((v - _N0) >> 31)
                    hivi = 1 + ((v - _PK0) >> 31)
                    loc = v - hivi * _PK0 - c * _BSP
                    ge0i = 1 + (loc >> 31)
                    lti = -((loc - _BSP) >> 31)
                    owni = reali * ge0i * lti
                    bufv[r, sl] = owni * (e + hivi * _N1P + 1) - 1
                    bufe[r, sl] = owni * (loc + 1) - 1
            return carry2
        lax.fori_loop(0, _SLAB, _tr, 0)

        def _chunk(j, carry2):
            if phase == 0:
                pltpu.sync_copy(tbl_hbm.at[bufv.at[j]], feat)
            else:
                pltpu.sync_copy(
                    tbl_hbm.at[plsc.Indices(bufv.at[j], ignored_value=-1)],
                    feat)
            pltpu.sync_copy(
                feat, acc.at[plsc.Indices(bufe.at[j], ignored_value=-1)],
                add=True)
            return carry2
        lax.fori_loop(0, _SLAB, _chunk, 0)
        return carry
    lax.fori_loop(0, nslab, _slab, 0)
    plsc.subcore_barrier()

    pltpu.sync_copy(acc.at[pl.ds(zbase, nacc)], parts.at[c, pl.ds(zbase, nacc)])


def _make_sc(phase, acc_rows):
    mesh = plsc.VectorSubcoreMesh(core_axis_name="c", subcore_axis_name="s",
                                  num_cores=2, num_subcores=16)
    out_type = [jax.ShapeDtypeStruct((2, acc_rows, 128), jnp.float32)]
    scratch = [
        pltpu.VMEM((_SLAB, _CW), jnp.int32),
        pltpu.VMEM((_SLAB, _CW), jnp.int32),
        pltpu.VMEM((_CW, 128), jnp.float32),
        pltpu.VMEM((_ZR, 128), jnp.float32),
        pltpu.VMEM_SHARED((acc_rows, 128), jnp.float32),
    ]
    body = functools.partial(_sc_body, phase=phase)
    return pl.kernel(body, out_type=out_type, mesh=mesh,
                     scratch_types=scratch)


def _deg_body(rows_hbm, deg0, deg1, buf, ones, zd, dacc):
    c = lax.axis_index("c")
    s = lax.axis_index("s")

    def _zo(r, carry):
        ones[pl.ds(16 * r, 16)] = jnp.ones((16,), jnp.float32)
        return carry
    lax.fori_loop(0, _CW // 16, _zo, 0)

    def _zd(r, carry):
        zd[pl.ds(16 * r, 16)] = jnp.zeros((16,), jnp.float32)
        return carry
    lax.fori_loop(0, 98, _zd, 0)
    nd = _N0P // 16
    pltpu.sync_copy(zd, dacc.at[pl.ds(s * nd, 1568)])
    pltpu.sync_copy(zd, dacc.at[pl.ds(s * nd + 1568, 1568)])
    plsc.subcore_barrier()

    wbase = (c * 16 + s) * 200

    def _slab(t, carry):
        pltpu.sync_copy(rows_hbm.at[pl.ds(wbase + t * _SLAB, _SLAB)], buf)

        def _chunk(j, carry2):
            pltpu.sync_copy(ones, dacc.at[buf.at[j]], add=True)
            return carry2
        lax.fori_loop(0, _SLAB, _chunk, 0)
        return carry
    lax.fori_loop(0, 25, _slab, 0)
    plsc.subcore_barrier()

    @pl.when(c == 0)
    def _():
        pltpu.sync_copy(dacc.at[pl.ds(s * nd, 1568)], zd)
        pltpu.sync_copy(zd, deg0.at[pl.ds(s * nd, 1568)])
        pltpu.sync_copy(dacc.at[pl.ds(s * nd + 1568, 1568)], zd)
        pltpu.sync_copy(zd, deg0.at[pl.ds(s * nd + 1568, 1568)])

    @pl.when(c == 1)
    def _():
        pltpu.sync_copy(dacc.at[pl.ds(s * nd, 1568)], zd)
        pltpu.sync_copy(zd, deg1.at[pl.ds(s * nd, 1568)])
        pltpu.sync_copy(dacc.at[pl.ds(s * nd + 1568, 1568)], zd)
        pltpu.sync_copy(zd, deg1.at[pl.ds(s * nd + 1568, 1568)])


def _make_deg():
    mesh = plsc.VectorSubcoreMesh(core_axis_name="c", subcore_axis_name="s",
                                  num_cores=2, num_subcores=16)
    out_type = [jax.ShapeDtypeStruct((_N0P,), jnp.float32),
                jax.ShapeDtypeStruct((_N0P,), jnp.float32)]
    scratch = [
        pltpu.VMEM((_SLAB, _CW), jnp.int32),
        pltpu.VMEM((_CW,), jnp.float32),
        pltpu.VMEM((1568,), jnp.float32),
        pltpu.VMEM_SHARED((_N0P,), jnp.float32),
    ]
    return pl.kernel(_deg_body, out_type=out_type, mesh=mesh,
                     scratch_types=scratch)


def _w_body(rows_hbm, cols_hbm, recip_hbm, w0, w1, bufr, bufc, vals, zw, wacc):
    c = lax.axis_index("c")
    s = lax.axis_index("s")

    def _zw(r, carry):
        zw[pl.ds(16 * r, 16)] = jnp.zeros((16,), jnp.float32)
        return carry
    lax.fori_loop(0, 98, _zw, 0)
    nw = _N1P // 16
    pltpu.sync_copy(zw, wacc.at[pl.ds(s * nw, nw)])
    plsc.subcore_barrier()

    wbase = (c * 16 + s) * 200

    def _slab(t, carry):
        start = wbase + t * _SLAB
        pltpu.sync_copy(rows_hbm.at[pl.ds(start, _SLAB)], bufr)
        pltpu.sync_copy(cols_hbm.at[pl.ds(start, _SLAB)], bufc)

        def _chunk(j, carry2):
            pltpu.sync_copy(recip_hbm.at[bufr.at[j]], vals)
            pltpu.sync_copy(vals, wacc.at[bufc.at[j]], add=True)
            return carry2
        lax.fori_loop(0, _SLAB, _chunk, 0)
        return carry
    lax.fori_loop(0, 25, _slab, 0)
    plsc.subcore_barrier()

    @pl.when(c == 0)
    def _():
        pltpu.sync_copy(wacc.at[pl.ds(s * nw, nw)], zw)
        pltpu.sync_copy(zw, w0.at[pl.ds(s * nw, nw)])

    @pl.when(c == 1)
    def _():
        pltpu.sync_copy(wacc.at[pl.ds(s * nw, nw)], zw)
        pltpu.sync_copy(zw, w1.at[pl.ds(s * nw, nw)])


def _make_w():
    mesh = plsc.VectorSubcoreMesh(core_axis_name="c", subcore_axis_name="s",
                                  num_cores=2, num_subcores=16)
    out_type = [jax.ShapeDtypeStruct((_N1P,), jnp.float32),
                jax.ShapeDtypeStruct((_N1P,), jnp.float32)]
    scratch = [
        pltpu.VMEM((_SLAB, _CW), jnp.int32),
        pltpu.VMEM((_SLAB, _CW), jnp.int32),
        pltpu.VMEM((_CW,), jnp.float32),
        pltpu.VMEM((_N1P // 16,), jnp.float32),
        pltpu.VMEM_SHARED((_N1P,), jnp.float32),
    ]
    return pl.kernel(_w_body, out_type=out_type, mesh=mesh,
                     scratch_types=scratch)


def _proj_body(x_ref, w_ref, b_ref, o_ref):
    th = pl.program_id(0)
    h = jnp.dot(x_ref[...], w_ref[...],
                preferred_element_type=jnp.float32) + b_ref[...]
    z = jnp.zeros_like(h)
    o_ref[...] = jnp.where(th == 0,
                           jnp.concatenate([h, z], axis=1),
                           jnp.concatenate([z, h], axis=1))[None]


def _proj_build(x0p, W, b):
    return pl.pallas_call(
        _proj_body,
        grid=(2, _N0P // 512),
        in_specs=[pl.BlockSpec((512, _D0), lambda t, g: (g, 0)),
                  pl.BlockSpec((_D0, _H), lambda t, g: (0, 0)),
                  pl.BlockSpec((1, _H), lambda t, g: (0, 0))],
        out_specs=pl.BlockSpec((1, 512, 128), lambda t, g: (t, g, 0)),
        out_shape=jax.ShapeDtypeStruct((2, _N0P, 128), jnp.float32),
    )(x0p, W, b)


def _comb_body(p_ref, o_ref):
    th = pl.program_id(0)
    ch = pl.program_id(1)
    S = p_ref[0] + p_ref[1]
    h = jnp.where(ch == 0, S[:, :_H], S[:, _H:])
    z = jnp.zeros_like(h)
    o_ref[...] = jnp.where(th == 0,
                           jnp.concatenate([h, z], axis=1),
                           jnp.concatenate([z, h], axis=1))[None]


def _combine_build(parts):
    blk = 448
    nb = _PK1 // blk
    return pl.pallas_call(
        _comb_body,
        grid=(2, 2, nb),
        in_specs=[pl.BlockSpec((2, blk, 128), lambda t, ch, g: (0, g, 0))],
        out_specs=pl.BlockSpec((1, blk, 128),
                               lambda t, ch, g: (t, ch * nb + g, 0)),
        out_shape=jax.ShapeDtypeStruct((2, _N1P, 128), jnp.float32),
    )(parts)


def _upd_body(pb_ref, t0_ref, d0_ref, d1_ref, o_ref, r_ref):
    th = pl.program_id(0)
    vh = pl.program_id(1) // 56
    S = pb_ref[0]
    m = jnp.where(vh == 0, S[:, :_H], S[:, _H:])
    d = jnp.maximum(d0_ref[...] + d1_ref[...], 1.0)
    h = t0_ref[0, :, :_H] + m / d
    z = jnp.zeros_like(h)
    o_ref[...] = jnp.where(th == 0,
                           jnp.concatenate([h, z], axis=1),
                           jnp.concatenate([z, h], axis=1))[None]
    r_ref[...] = 1.0 / d


def _update_build(partsb, T0, deg0, deg1):
    nbs = _BSP // 448

    def _pb_idx(t, g):
        gq = g % 56
        part = gq // nbs
        return (part, gq - part * nbs, 0)

    dspec = pl.BlockSpec((448, 1), lambda t, g: (g, 0))
    return pl.pallas_call(
        _upd_body,
        grid=(2, 112),
        in_specs=[pl.BlockSpec((1, 448, 128), _pb_idx),
                  pl.BlockSpec((1, 448, 128), lambda t, g: (0, g, 0)),
                  dspec, dspec],
        out_specs=[pl.BlockSpec((1, 448, 128), lambda t, g: (t, g, 0)),
                   pl.BlockSpec((448, 1), lambda t, g: (g, 0))],
        out_shape=[jax.ShapeDtypeStruct((2, _N0P, 128), jnp.float32),
                   jax.ShapeDtypeStruct((_N0P, 1), jnp.float32)],
    )(partsb, T0, deg0.reshape(_N0P, 1), deg1.reshape(_N0P, 1))


def _fin_body(t0_ref, t1_ref, we0_ref, we1_ref, w0_ref, b0_ref, w1_ref,
              b1_ref, o_ref, acc0, acc1, accw):
    g = pl.program_id(0)

    @pl.when(g == 0)
    def _():
        acc0[...] = jnp.zeros_like(acc0)
        acc1[...] = jnp.zeros_like(acc1)
        accw[...] = jnp.zeros_like(accw)

    b0r = _N0P // 32
    r0 = lax.broadcasted_iota(jnp.int32, (b0r, 128), 0) + g * b0r
    acc0[...] += jnp.sum(jnp.where(r0 < _N0, t0_ref[0], 0.0),
                         axis=0, keepdims=True)
    acc1[...] += jnp.sum(t1_ref[0], axis=0, keepdims=True)
    we = we0_ref[...] + we1_ref[...]
    accw[...] += jnp.sum(we * t1_ref[0], axis=0, keepdims=True)

    @pl.when(g == 31)
    def _():
        s0 = (acc0[:, :_H] + accw[:, :_H]) * (1.0 / _N0)
        s1 = acc1[:, :_H] * (1.0 / _N1)
        o_ref[...] = (jnp.dot(s0, w0_ref[...],
                              preferred_element_type=jnp.float32) + b0_ref[...]
                      + jnp.dot(s1, w1_ref[...],
                                preferred_element_type=jnp.float32)
                      + b1_ref[...])


def _final(T0, T1, we0, we1, W0o, b0o, W1o, b1o):
    wspec = pl.BlockSpec((_N1P // 32, 1), lambda g: (g, 0))
    return pl.pallas_call(
        _fin_body,
        grid=(32,),
        in_specs=[pl.BlockSpec((1, _N0P // 32, 128), lambda g: (0, g, 0)),
                  pl.BlockSpec((1, _N1P // 32, 128), lambda g: (0, g, 0)),
                  wspec, wspec,
                  pl.BlockSpec((_H, 1), lambda g: (0, 0)),
                  pl.BlockSpec((1, 1), lambda g: (0, 0)),
                  pl.BlockSpec((_H, 1), lambda g: (0, 0)),
                  pl.BlockSpec((1, 1), lambda g: (0, 0))],
        out_specs=pl.BlockSpec((1, 1), lambda g: (0, 0)),
        out_shape=jax.ShapeDtypeStruct((1, 1), jnp.float32),
        scratch_shapes=[pltpu.VMEM((1, 128), jnp.float32),
                        pltpu.VMEM((1, 128), jnp.float32),
                        pltpu.VMEM((1, 128), jnp.float32)],
    )(T0, T1, we0.reshape(_N1P, 1), we1.reshape(_N1P, 1),
      W0o, b0o, W1o, b1o)


def kernel(x_0, x_1, incidence_rows, incidence_cols,
           W0_in, b0_in, W1_in, b1_in, W0_out, b0_out, W1_out, b1_out):
    del x_1, W1_in, b1_in
    npad = _NNZP - _NNZ
    rows_p = jnp.concatenate(
        [incidence_rows, jnp.full((npad,), _N0, jnp.int32)]).reshape(
        _IDXR, _CW)
    cols_p = jnp.concatenate(
        [incidence_cols, jnp.full((npad,), _N1, jnp.int32)]).reshape(
        _IDXR, _CW)
    x0p = jnp.concatenate([x_0, jnp.zeros((_N0P - _N0, _D0), jnp.float32)])

    T0 = _proj_build(x0p, W0_in, b0_in.reshape(1, _H))
    deg0, deg1 = _make_deg()(rows_p)

    phase_a = _make_sc(phase=0, acc_rows=_PK1)
    phase_b = _make_sc(phase=1, acc_rows=_BSP)

    parts1, = phase_a(rows_p, cols_p, T0.reshape(2 * _N0P, 128))
    T1 = _combine_build(parts1)
    partsb, = phase_b(rows_p, cols_p, T1.reshape(2 * _N1P, 128))
    T0, recip = _update_build(partsb, T0, deg0, deg1)

    we0, we1 = _make_w()(rows_p, cols_p, recip.reshape(_N0P))
    parts2, = phase_a(rows_p, cols_p, T0.reshape(2 * _N0P, 128))
    T1 = _combine_build(parts2)

    out = _final(T0, T1, we0, we1, W0_out, b0_out.reshape(1, 1),
                 W1_out, b1_out.reshape(1, 1))
    return out.reshape(1)
